# baseline (speedup 1.0000x reference)
import jax
import jax.numpy as jnp
from jax.experimental import pallas as pl

_K = 3
_CHUNK = 1000


def kernel(x, pos, pos_up, batch, batch_up):
    x2 = jnp.sum(pos * pos, axis=1)
    def body(y_chunk):
        d = jnp.sum(y_chunk * y_chunk, axis=1, keepdims=True) - 2.0 * (y_chunk @ pos.T) + x2[None, :]
        vals, idx = jax.lax.top_k(-d, _K)
        return -vals, idx
    M, D = pos_up.shape
    ych = pos_up.reshape(M // _CHUNK, _CHUNK, D)
    dists, idxs = jax.lax.map(body, ych)
    sqd = jnp.maximum(dists.reshape(M, _K), 1e-16)
    idx = idxs.reshape(M, _K)
    w = 1.0 / sqd
    gathered = x[idx]
    num = jnp.sum(gathered * w[..., None], axis=1)
    den = jnp.sum(w, axis=1, keepdims=True)
    return num / den


# speedup vs baseline: 2.8552x; 2.8552x over previous
import functools

import jax
import jax.numpy as jnp
from jax import lax
from jax.experimental import pallas as pl
from jax.experimental.pallas import tpu as pltpu
from jax.experimental.pallas import tpu_sc as plsc

_BIG = 3.0e38
_PADKEY = 1.0e9


def _ceil_to(a, b):
    return (a + b - 1) // b * b


def _topk_kernel(nchunk, t_cols, bq, y_ref, padj_ref, x2_ref, wn_ref,
                 idx_ref, d_ref, v0_ref, v1_ref, v2_ref, i0_ref, i1_ref,
                 i2_ref):
    groups = t_cols // 128
    qsubs = bq // 8
    v0_ref[...] = jnp.full((bq, 128), _BIG, jnp.float32)
    v1_ref[...] = jnp.full((bq, 128), _BIG, jnp.float32)
    v2_ref[...] = jnp.full((bq, 128), _BIG, jnp.float32)
    i0_ref[...] = jnp.zeros((bq, 128), jnp.int32)
    i1_ref[...] = jnp.zeros((bq, 128), jnp.int32)
    i2_ref[...] = jnp.zeros((bq, 128), jnp.int32)
    ya = y_ref[...]

    lane = lax.broadcasted_iota(jnp.int32, (8, 128), 1)

    def chunk_body(j, _):
        d_ref[...] = lax.dot_general(
            ya[:, :3], padj_ref[j], (((1,), (0,)), ((), ())),
            preferred_element_type=jnp.float32)
        base = j * t_cols

        def qsub_body(q, _):
            rs = pl.ds(q * 8, 8)
            y2b = jnp.broadcast_to(y_ref[rs, 3:4], (8, 128))
            v0 = v0_ref[rs, :]
            v1 = v1_ref[rs, :]
            v2 = v2_ref[rs, :]
            i0 = i0_ref[rs, :]
            i1 = i1_ref[rs, :]
            i2 = i2_ref[rs, :]
            for g in range(groups):
                pg = d_ref[rs, g * 128:(g + 1) * 128]
                x2g = jnp.broadcast_to(
                    x2_ref[j, 0:1, g * 128:(g + 1) * 128], (8, 128))
                dv = (y2b - 2.0 * pg) + x2g
                ii = lane + (base + g * 128)
                c0 = dv < v0
                c1 = dv < v1
                c2 = dv < v2
                l0 = jnp.maximum(v0, dv)
                nv0 = jnp.minimum(v0, dv)
                l1 = jnp.maximum(v1, l0)
                nv1 = jnp.minimum(v1, l0)
                nv2 = jnp.minimum(v2, l1)
                ni0 = jnp.where(c0, ii, i0)
                ni1 = jnp.where(c1, jnp.where(c0, i0, ii), i1)
                ni2 = jnp.where(c2, jnp.where(c1, i1, ii), i2)
                v0, v1, v2, i0, i1, i2 = nv0, nv1, nv2, ni0, ni1, ni2
            v0_ref[rs, :] = v0
            v1_ref[rs, :] = v1
            v2_ref[rs, :] = v2
            i0_ref[rs, :] = i0
            i1_ref[rs, :] = i1
            i2_ref[rs, :] = i2
            return 0

        lax.fori_loop(0, qsubs, qsub_body, 0)
        return 0

    lax.fori_loop(0, nchunk, chunk_body, 0)

    vals = jnp.concatenate([v0_ref[...], v1_ref[...], v2_ref[...]], axis=1)
    inds = jnp.concatenate([i0_ref[...], i1_ref[...], i2_ref[...]], axis=1)
    keys, picks = [], []
    for _ in range(3):
        m = jnp.min(vals, axis=1, keepdims=True)
        sel = vals == m
        ci = jnp.min(jnp.where(sel, inds, jnp.int32(2**31 - 1)),
                     axis=1, keepdims=True)
        keys.append(m)
        picks.append(ci)
        vals = jnp.where(sel & (inds == ci), _BIG, vals)

    ws = [1.0 / jnp.maximum(k, jnp.float32(1e-16)) for k in keys]
    wsum = ws[0] + ws[1] + ws[2]
    wn_ref[...] = jnp.concatenate([w / wsum for w in ws], axis=1)
    idx_ref[...] = jnp.concatenate(picks, axis=1)


def _run_topk(pos, pos_up, bq=512, t_cols=3584):
    n = pos.shape[0]
    m = pos_up.shape[0]
    np_ = _ceil_to(n, t_cols)
    nchunk = np_ // t_cols
    m_pad = _ceil_to(m, bq)
    grid = m_pad // bq

    pos_b = pos.astype(jnp.bfloat16).astype(jnp.float32)
    posup_b = pos_up.astype(jnp.bfloat16).astype(jnp.float32)
    x2 = jnp.sum(pos * pos, axis=1)
    padj = jnp.concatenate(
        [pos_b.T, jnp.zeros((3, np_ - n), jnp.float32)], axis=1)
    padj3 = padj.reshape(3, nchunk, t_cols).transpose(1, 0, 2)
    x2p = jnp.concatenate(
        [x2, jnp.full((np_ - n,), _PADKEY, jnp.float32)])
    x23 = x2p.reshape(nchunk, 1, t_cols)

    y2 = jnp.sum(pos_up * pos_up, axis=1, keepdims=True)
    ya = jnp.concatenate([posup_b, y2], axis=1)
    ya = jnp.concatenate(
        [ya, jnp.zeros((m_pad - m, 4), jnp.float32)], axis=0)

    wn, idx = pl.pallas_call(
        functools.partial(_topk_kernel, nchunk, t_cols, bq),
        grid=(grid,),
        in_specs=[
            pl.BlockSpec((bq, 4), lambda i: (i, 0)),
            pl.BlockSpec((nchunk, 3, t_cols), lambda i: (0, 0, 0)),
            pl.BlockSpec((nchunk, 1, t_cols), lambda i: (0, 0, 0)),
        ],
        out_specs=[
            pl.BlockSpec((bq, 3), lambda i: (i, 0)),
            pl.BlockSpec((bq, 3), lambda i: (i, 0)),
        ],
        out_shape=[
            jax.ShapeDtypeStruct((m_pad, 3), jnp.float32),
            jax.ShapeDtypeStruct((m_pad, 3), jnp.int32),
        ],
        scratch_shapes=[
            pltpu.VMEM((bq, t_cols), jnp.float32),
            pltpu.VMEM((bq, 128), jnp.float32),
            pltpu.VMEM((bq, 128), jnp.float32),
            pltpu.VMEM((bq, 128), jnp.float32),
            pltpu.VMEM((bq, 128), jnp.int32),
            pltpu.VMEM((bq, 128), jnp.int32),
            pltpu.VMEM((bq, 128), jnp.int32),
        ],
    )(ya, padj3, x23)
    return wn, idx, m_pad


def _interp_sc(x, i0, i1, i2, w0, w1, w2, batch_rows=112):
    m_pad = i0.shape[0]
    info = plsc.get_sparse_core_info()
    nworkers = info.num_cores * info.num_subcores
    per_w = m_pad // nworkers
    nb = per_w // batch_rows
    d = x.shape[1]
    nc = info.num_cores

    mesh = plsc.VectorSubcoreMesh(core_axis_name="c", subcore_axis_name="s")

    @functools.partial(
        pl.kernel, mesh=mesh,
        out_type=jax.ShapeDtypeStruct((m_pad, d), jnp.float32),
        scratch_types=[
            pltpu.VMEM((batch_rows,), jnp.int32),
            pltpu.VMEM((batch_rows,), jnp.int32),
            pltpu.VMEM((batch_rows,), jnp.int32),
            pltpu.VMEM((batch_rows, 16), jnp.float32),
            pltpu.VMEM((batch_rows, 16), jnp.float32),
            pltpu.VMEM((batch_rows, 16), jnp.float32),
            pltpu.VMEM((batch_rows, d), jnp.float32),
            pltpu.VMEM((batch_rows, d), jnp.float32),
            pltpu.VMEM((batch_rows, d), jnp.float32),
            pltpu.VMEM((batch_rows, d), jnp.float32),
            pltpu.SemaphoreType.DMA,
        ],
    )
    def interp(x_hbm, i0_hbm, i1_hbm, i2_hbm, w0_hbm, w1_hbm, w2_hbm,
               out_hbm, i0v, i1v, i2v, w0v, w1v, w2v, r0v, r1v, r2v,
               outv, sem):
        wid = lax.axis_index("s") * nc + lax.axis_index("c")
        base = wid * per_w

        def batch_body(b, _):
            off = base + b * batch_rows
            sl = pl.ds(off, batch_rows)
            pltpu.sync_copy(i0_hbm.at[sl], i0v)
            pltpu.sync_copy(i1_hbm.at[sl], i1v)
            pltpu.sync_copy(i2_hbm.at[sl], i2v)
            pltpu.sync_copy(w0_hbm.at[sl], w0v)
            pltpu.sync_copy(w1_hbm.at[sl], w1v)
            pltpu.sync_copy(w2_hbm.at[sl], w2v)
            c0 = pltpu.async_copy(x_hbm.at[i0v], r0v, sem)
            c1 = pltpu.async_copy(x_hbm.at[i1v], r1v, sem)
            c2 = pltpu.async_copy(x_hbm.at[i2v], r2v, sem)
            c0.wait()
            c1.wait()
            c2.wait()

            def row_body(r, _):
                w0s = w0v[r, :]
                w1s = w1v[r, :]
                w2s = w2v[r, :]
                for c in range(d // 16):
                    s = pl.ds(c * 16, 16)
                    outv[r, s] = (r0v[r, s] * w0s + r1v[r, s] * w1s
                                  + r2v[r, s] * w2s)
                return 0

            lax.fori_loop(0, batch_rows, row_body, 0)
            pltpu.sync_copy(outv, out_hbm.at[sl])
            return 0

        lax.fori_loop(0, nb, batch_body, 0)

    return interp(x, i0, i1, i2, w0, w1, w2)


def kernel(x, pos, pos_up, batch, batch_up):
    m = pos_up.shape[0]
    wn, idx, m_pad = _run_topk(pos, pos_up)
    i0 = idx[:, 0]
    i1 = idx[:, 1]
    i2 = idx[:, 2]
    m_pad = i0.shape[0]
    w0 = jnp.broadcast_to(wn[:, 0:1], (m_pad, 16))
    w1 = jnp.broadcast_to(wn[:, 1:2], (m_pad, 16))
    w2 = jnp.broadcast_to(wn[:, 2:3], (m_pad, 16))
    out = _interp_sc(x, i0, i1, i2, w0, w1, w2)
    return out[:m]


# speedup vs baseline: 4.4514x; 1.5591x over previous
import functools

import jax
import jax.numpy as jnp
from jax import lax
from jax.experimental import pallas as pl
from jax.experimental.pallas import tpu as pltpu
from jax.experimental.pallas import tpu_sc as plsc

_BIG = 3.0e38
_PADKEY = 1.0e9


def _ceil_to(a, b):
    return (a + b - 1) // b * b


_ROWS = 4


def _topk_kernel(nchunk, t_cols, bq, y_ref, padj_ref, x2_ref, wn_ref,
                 idx_ref, d_ref, v0_ref, v1_ref, v2_ref, i0_ref, i1_ref,
                 i2_ref):
    groups = t_cols // 128
    qsubs = bq // (8 * _ROWS)
    v0_ref[...] = jnp.full((bq, 128), _BIG, jnp.float32)
    v1_ref[...] = jnp.full((bq, 128), _BIG, jnp.float32)
    v2_ref[...] = jnp.full((bq, 128), _BIG, jnp.float32)
    i0_ref[...] = jnp.zeros((bq, 128), jnp.int32)
    i1_ref[...] = jnp.zeros((bq, 128), jnp.int32)
    i2_ref[...] = jnp.zeros((bq, 128), jnp.int32)
    ya = y_ref[...]

    def chunk_body(j, _):
        d_ref[...] = lax.dot_general(
            ya[:, :3], padj_ref[j], (((1,), (0,)), ((), ())),
            preferred_element_type=jnp.float32)
        base = j * t_cols

        def qsub_body(q, _):
            q0 = q * (8 * _ROWS)
            rss = [pl.ds(q0 + 8 * r, 8) for r in range(_ROWS)]
            st = []
            y2b = []
            for rs in rss:
                st.append([v0_ref[rs, :], v1_ref[rs, :], v2_ref[rs, :],
                           i0_ref[rs, :], i1_ref[rs, :], i2_ref[rs, :]])
                y2b.append(jnp.broadcast_to(y_ref[rs, 3:4], (8, 128)))
            for g in range(groups):
                gb = base + g * 128
                x2g = jnp.broadcast_to(
                    x2_ref[j, 0:1, g * 128:(g + 1) * 128], (8, 128))
                for r in range(_ROWS):
                    v0, v1, v2, i0, i1, i2 = st[r]
                    pg = d_ref[rss[r], g * 128:(g + 1) * 128]
                    dv = (y2b[r] + pg) + x2g
                    c0 = dv < v0
                    c1 = dv < v1
                    c2 = dv < v2
                    l0 = jnp.maximum(v0, dv)
                    nv0 = jnp.minimum(v0, dv)
                    l1 = jnp.maximum(v1, l0)
                    nv1 = jnp.minimum(v1, l0)
                    nv2 = jnp.minimum(v2, l1)
                    ni0 = jnp.where(c0, gb, i0)
                    ni1 = jnp.where(c1, jnp.where(c0, i0, gb), i1)
                    ni2 = jnp.where(c2, jnp.where(c1, i1, gb), i2)
                    st[r] = [nv0, nv1, nv2, ni0, ni1, ni2]
            for r, rs in enumerate(rss):
                v0_ref[rs, :] = st[r][0]
                v1_ref[rs, :] = st[r][1]
                v2_ref[rs, :] = st[r][2]
                i0_ref[rs, :] = st[r][3]
                i1_ref[rs, :] = st[r][4]
                i2_ref[rs, :] = st[r][5]
            return 0

        lax.fori_loop(0, qsubs, qsub_body, 0)
        return 0

    lax.fori_loop(0, nchunk, chunk_body, 0)

    laneb = lax.broadcasted_iota(jnp.int32, (bq, 128), 1)
    vals = jnp.concatenate([v0_ref[...], v1_ref[...], v2_ref[...]], axis=1)
    inds = jnp.concatenate([i0_ref[...] + laneb, i1_ref[...] + laneb,
                            i2_ref[...] + laneb], axis=1)
    keys, picks = [], []
    for _ in range(3):
        m = jnp.min(vals, axis=1, keepdims=True)
        sel = vals == m
        ci = jnp.min(jnp.where(sel, inds, jnp.int32(2**31 - 1)),
                     axis=1, keepdims=True)
        keys.append(m)
        picks.append(ci)
        vals = jnp.where(sel & (inds == ci), _BIG, vals)

    ws = [1.0 / jnp.maximum(k, jnp.float32(1e-16)) for k in keys]
    wsum = ws[0] + ws[1] + ws[2]
    wn_ref[...] = jnp.concatenate([w / wsum for w in ws], axis=1)
    idx_ref[...] = jnp.concatenate(picks, axis=1)


def _run_topk(pos, pos_up, bq=512, t_cols=3584):
    n = pos.shape[0]
    m = pos_up.shape[0]
    np_ = _ceil_to(n, t_cols)
    nchunk = np_ // t_cols
    m_pad = _ceil_to(m, bq)
    grid = m_pad // bq

    pos_b = pos.astype(jnp.bfloat16).astype(jnp.float32)
    posup_b = pos_up.astype(jnp.bfloat16).astype(jnp.float32)
    x2 = jnp.sum(pos * pos, axis=1)
    padj = jnp.concatenate(
        [-2.0 * pos_b.T, jnp.zeros((3, np_ - n), jnp.float32)], axis=1)
    padj3 = padj.reshape(3, nchunk, t_cols).transpose(1, 0, 2)
    x2p = jnp.concatenate(
        [x2, jnp.full((np_ - n,), _PADKEY, jnp.float32)])
    x23 = x2p.reshape(nchunk, 1, t_cols)

    y2 = jnp.sum(pos_up * pos_up, axis=1, keepdims=True)
    ya = jnp.concatenate([posup_b, y2], axis=1)
    ya = jnp.concatenate(
        [ya, jnp.zeros((m_pad - m, 4), jnp.float32)], axis=0)

    wn, idx = pl.pallas_call(
        functools.partial(_topk_kernel, nchunk, t_cols, bq),
        grid=(grid,),
        in_specs=[
            pl.BlockSpec((bq, 4), lambda i: (i, 0)),
            pl.BlockSpec((nchunk, 3, t_cols), lambda i: (0, 0, 0)),
            pl.BlockSpec((nchunk, 1, t_cols), lambda i: (0, 0, 0)),
        ],
        out_specs=[
            pl.BlockSpec((bq, 3), lambda i: (i, 0)),
            pl.BlockSpec((bq, 3), lambda i: (i, 0)),
        ],
        out_shape=[
            jax.ShapeDtypeStruct((m_pad, 3), jnp.float32),
            jax.ShapeDtypeStruct((m_pad, 3), jnp.int32),
        ],
        scratch_shapes=[
            pltpu.VMEM((bq, t_cols), jnp.float32),
            pltpu.VMEM((bq, 128), jnp.float32),
            pltpu.VMEM((bq, 128), jnp.float32),
            pltpu.VMEM((bq, 128), jnp.float32),
            pltpu.VMEM((bq, 128), jnp.int32),
            pltpu.VMEM((bq, 128), jnp.int32),
            pltpu.VMEM((bq, 128), jnp.int32),
        ],
    )(ya, padj3, x23)
    return wn, idx, m_pad


def _interp_sc(x, i0, i1, i2, w0, w1, w2, batch_rows=112):
    m_pad = i0.shape[0]
    info = plsc.get_sparse_core_info()
    nworkers = info.num_cores * info.num_subcores
    per_w = m_pad // nworkers
    nb = per_w // batch_rows
    d = x.shape[1]
    nc = info.num_cores

    mesh = plsc.VectorSubcoreMesh(core_axis_name="c", subcore_axis_name="s")

    @functools.partial(
        pl.kernel, mesh=mesh,
        out_type=jax.ShapeDtypeStruct((m_pad, d), jnp.float32),
        scratch_types=[
            pltpu.VMEM((batch_rows,), jnp.int32),
            pltpu.VMEM((batch_rows,), jnp.int32),
            pltpu.VMEM((batch_rows,), jnp.int32),
            pltpu.VMEM((batch_rows, 16), jnp.float32),
            pltpu.VMEM((batch_rows, 16), jnp.float32),
            pltpu.VMEM((batch_rows, 16), jnp.float32),
            pltpu.VMEM((batch_rows, d), jnp.float32),
            pltpu.VMEM((batch_rows, d), jnp.float32),
            pltpu.VMEM((batch_rows, d), jnp.float32),
            pltpu.VMEM((batch_rows, d), jnp.float32),
            pltpu.SemaphoreType.DMA,
        ],
    )
    def interp(x_hbm, i0_hbm, i1_hbm, i2_hbm, w0_hbm, w1_hbm, w2_hbm,
               out_hbm, i0v, i1v, i2v, w0v, w1v, w2v, r0v, r1v, r2v,
               outv, sem):
        wid = lax.axis_index("s") * nc + lax.axis_index("c")
        base = wid * per_w

        def batch_body(b, _):
            off = base + b * batch_rows
            sl = pl.ds(off, batch_rows)
            pltpu.sync_copy(i0_hbm.at[sl], i0v)
            pltpu.sync_copy(i1_hbm.at[sl], i1v)
            pltpu.sync_copy(i2_hbm.at[sl], i2v)
            pltpu.sync_copy(w0_hbm.at[sl], w0v)
            pltpu.sync_copy(w1_hbm.at[sl], w1v)
            pltpu.sync_copy(w2_hbm.at[sl], w2v)
            c0 = pltpu.async_copy(x_hbm.at[i0v], r0v, sem)
            c1 = pltpu.async_copy(x_hbm.at[i1v], r1v, sem)
            c2 = pltpu.async_copy(x_hbm.at[i2v], r2v, sem)
            c0.wait()
            c1.wait()
            c2.wait()

            def row_body(r, _):
                w0s = w0v[r, :]
                w1s = w1v[r, :]
                w2s = w2v[r, :]
                for c in range(d // 16):
                    s = pl.ds(c * 16, 16)
                    outv[r, s] = (r0v[r, s] * w0s + r1v[r, s] * w1s
                                  + r2v[r, s] * w2s)
                return 0

            lax.fori_loop(0, batch_rows, row_body, 0)
            pltpu.sync_copy(outv, out_hbm.at[sl])
            return 0

        lax.fori_loop(0, nb, batch_body, 0)

    return interp(x, i0, i1, i2, w0, w1, w2)


def kernel(x, pos, pos_up, batch, batch_up):
    m = pos_up.shape[0]
    wn, idx, m_pad = _run_topk(pos, pos_up)
    i0 = idx[:, 0]
    i1 = idx[:, 1]
    i2 = idx[:, 2]
    m_pad = i0.shape[0]
    w0 = jnp.broadcast_to(wn[:, 0:1], (m_pad, 16))
    w1 = jnp.broadcast_to(wn[:, 1:2], (m_pad, 16))
    w2 = jnp.broadcast_to(wn[:, 2:3], (m_pad, 16))
    out = _interp_sc(x, i0, i1, i2, w0, w1, w2)
    return out[:m]


# speedup vs baseline: 4.4660x; 1.0033x over previous
import functools

import jax
import jax.numpy as jnp
from jax import lax
from jax.experimental import pallas as pl
from jax.experimental.pallas import tpu as pltpu
from jax.experimental.pallas import tpu_sc as plsc

_BIG = 3.0e38
_PADKEY = 1.0e9


def _ceil_to(a, b):
    return (a + b - 1) // b * b


_ROWS = 4


def _topk_kernel(nchunk, t_cols, bq, y_ref, padj_ref, x2_ref, wn_ref,
                 idx_ref, d_ref, v0_ref, v1_ref, v2_ref, i0_ref, i1_ref,
                 i2_ref):
    groups = t_cols // 128
    qsubs = bq // (8 * _ROWS)
    v0_ref[...] = jnp.full((bq, 128), _BIG, jnp.float32)
    v1_ref[...] = jnp.full((bq, 128), _BIG, jnp.float32)
    v2_ref[...] = jnp.full((bq, 128), _BIG, jnp.float32)
    i0_ref[...] = jnp.zeros((bq, 128), jnp.int32)
    i1_ref[...] = jnp.zeros((bq, 128), jnp.int32)
    i2_ref[...] = jnp.zeros((bq, 128), jnp.int32)
    ya = y_ref[...]

    def dot_into(j, slot):
        d_ref[slot] = lax.dot_general(
            ya[:, :3], padj_ref[j], (((1,), (0,)), ((), ())),
            preferred_element_type=jnp.float32)

    dot_into(0, 0)

    def chunk_body(j, _):
        nxt = jnp.minimum(j + 1, nchunk - 1)

        @pl.when(j + 1 < nchunk)
        def _():
            dot_into(nxt, (j + 1) % 2)

        slot = j % 2
        base = j * t_cols

        def qsub_body(q, _):
            q0 = q * (8 * _ROWS)
            rss = [pl.ds(q0 + 8 * r, 8) for r in range(_ROWS)]
            st = []
            y2b = []
            for rs in rss:
                st.append([v0_ref[rs, :], v1_ref[rs, :], v2_ref[rs, :],
                           i0_ref[rs, :], i1_ref[rs, :], i2_ref[rs, :]])
                y2b.append(jnp.broadcast_to(y_ref[rs, 3:4], (8, 128)))
            for g in range(groups):
                gb = base + g * 128
                x2g = jnp.broadcast_to(
                    x2_ref[j, 0:1, g * 128:(g + 1) * 128], (8, 128))
                for r in range(_ROWS):
                    v0, v1, v2, i0, i1, i2 = st[r]
                    pg = d_ref[slot, rss[r], g * 128:(g + 1) * 128]
                    dv = (y2b[r] + pg) + x2g
                    c0 = dv < v0
                    c1 = dv < v1
                    c2 = dv < v2
                    l0 = jnp.maximum(v0, dv)
                    nv0 = jnp.minimum(v0, dv)
                    l1 = jnp.maximum(v1, l0)
                    nv1 = jnp.minimum(v1, l0)
                    nv2 = jnp.minimum(v2, l1)
                    ni0 = jnp.where(c0, gb, i0)
                    ni1 = jnp.where(c1, jnp.where(c0, i0, gb), i1)
                    ni2 = jnp.where(c2, jnp.where(c1, i1, gb), i2)
                    st[r] = [nv0, nv1, nv2, ni0, ni1, ni2]
            for r, rs in enumerate(rss):
                v0_ref[rs, :] = st[r][0]
                v1_ref[rs, :] = st[r][1]
                v2_ref[rs, :] = st[r][2]
                i0_ref[rs, :] = st[r][3]
                i1_ref[rs, :] = st[r][4]
                i2_ref[rs, :] = st[r][5]
            return 0

        lax.fori_loop(0, qsubs, qsub_body, 0)
        return 0

    lax.fori_loop(0, nchunk, chunk_body, 0)

    laneb = lax.broadcasted_iota(jnp.int32, (bq, 128), 1)
    vals = jnp.concatenate([v0_ref[...], v1_ref[...], v2_ref[...]], axis=1)
    inds = jnp.concatenate([i0_ref[...] + laneb, i1_ref[...] + laneb,
                            i2_ref[...] + laneb], axis=1)
    keys, picks = [], []
    for _ in range(3):
        m = jnp.min(vals, axis=1, keepdims=True)
        sel = vals == m
        ci = jnp.min(jnp.where(sel, inds, jnp.int32(2**31 - 1)),
                     axis=1, keepdims=True)
        keys.append(m)
        picks.append(ci)
        vals = jnp.where(sel & (inds == ci), _BIG, vals)

    ws = [1.0 / jnp.maximum(k, jnp.float32(1e-16)) for k in keys]
    wsum = ws[0] + ws[1] + ws[2]
    wn_ref[...] = jnp.concatenate([w / wsum for w in ws], axis=1)
    idx_ref[...] = jnp.concatenate(picks, axis=1)


def _run_topk(pos, pos_up, bq=512, t_cols=3584):
    n = pos.shape[0]
    m = pos_up.shape[0]
    np_ = _ceil_to(n, t_cols)
    nchunk = np_ // t_cols
    m_pad = _ceil_to(m, bq)
    grid = m_pad // bq

    pos_b = pos.astype(jnp.bfloat16).astype(jnp.float32)
    posup_b = pos_up.astype(jnp.bfloat16).astype(jnp.float32)
    x2 = jnp.sum(pos * pos, axis=1)
    padj = jnp.concatenate(
        [-2.0 * pos_b.T, jnp.zeros((3, np_ - n), jnp.float32)], axis=1)
    padj3 = padj.reshape(3, nchunk, t_cols).transpose(1, 0, 2)
    x2p = jnp.concatenate(
        [x2, jnp.full((np_ - n,), _PADKEY, jnp.float32)])
    x23 = x2p.reshape(nchunk, 1, t_cols)

    y2 = jnp.sum(pos_up * pos_up, axis=1, keepdims=True)
    ya = jnp.concatenate([posup_b, y2], axis=1)
    ya = jnp.concatenate(
        [ya, jnp.zeros((m_pad - m, 4), jnp.float32)], axis=0)

    wn, idx = pl.pallas_call(
        functools.partial(_topk_kernel, nchunk, t_cols, bq),
        grid=(grid,),
        in_specs=[
            pl.BlockSpec((bq, 4), lambda i: (i, 0)),
            pl.BlockSpec((nchunk, 3, t_cols), lambda i: (0, 0, 0)),
            pl.BlockSpec((nchunk, 1, t_cols), lambda i: (0, 0, 0)),
        ],
        out_specs=[
            pl.BlockSpec((bq, 3), lambda i: (i, 0)),
            pl.BlockSpec((bq, 3), lambda i: (i, 0)),
        ],
        out_shape=[
            jax.ShapeDtypeStruct((m_pad, 3), jnp.float32),
            jax.ShapeDtypeStruct((m_pad, 3), jnp.int32),
        ],
        scratch_shapes=[
            pltpu.VMEM((2, bq, t_cols), jnp.float32),
            pltpu.VMEM((bq, 128), jnp.float32),
            pltpu.VMEM((bq, 128), jnp.float32),
            pltpu.VMEM((bq, 128), jnp.float32),
            pltpu.VMEM((bq, 128), jnp.int32),
            pltpu.VMEM((bq, 128), jnp.int32),
            pltpu.VMEM((bq, 128), jnp.int32),
        ],
    )(ya, padj3, x23)
    return wn, idx, m_pad


def _interp_sc(x, i0, i1, i2, w0, w1, w2, batch_rows=112):
    m_pad = i0.shape[0]
    info = plsc.get_sparse_core_info()
    nworkers = info.num_cores * info.num_subcores
    per_w = m_pad // nworkers
    nb = per_w // batch_rows
    d = x.shape[1]
    nc = info.num_cores

    mesh = plsc.VectorSubcoreMesh(core_axis_name="c", subcore_axis_name="s")

    @functools.partial(
        pl.kernel, mesh=mesh,
        out_type=jax.ShapeDtypeStruct((m_pad, d), jnp.float32),
        scratch_types=[
            pltpu.VMEM((batch_rows,), jnp.int32),
            pltpu.VMEM((batch_rows,), jnp.int32),
            pltpu.VMEM((batch_rows,), jnp.int32),
            pltpu.VMEM((batch_rows, 16), jnp.float32),
            pltpu.VMEM((batch_rows, 16), jnp.float32),
            pltpu.VMEM((batch_rows, 16), jnp.float32),
            pltpu.VMEM((batch_rows, d), jnp.float32),
            pltpu.VMEM((batch_rows, d), jnp.float32),
            pltpu.VMEM((batch_rows, d), jnp.float32),
            pltpu.VMEM((batch_rows, d), jnp.float32),
            pltpu.SemaphoreType.DMA,
        ],
    )
    def interp(x_hbm, i0_hbm, i1_hbm, i2_hbm, w0_hbm, w1_hbm, w2_hbm,
               out_hbm, i0v, i1v, i2v, w0v, w1v, w2v, r0v, r1v, r2v,
               outv, sem):
        wid = lax.axis_index("s") * nc + lax.axis_index("c")
        base = wid * per_w

        def batch_body(b, _):
            off = base + b * batch_rows
            sl = pl.ds(off, batch_rows)
            pltpu.sync_copy(i0_hbm.at[sl], i0v)
            pltpu.sync_copy(i1_hbm.at[sl], i1v)
            pltpu.sync_copy(i2_hbm.at[sl], i2v)
            pltpu.sync_copy(w0_hbm.at[sl], w0v)
            pltpu.sync_copy(w1_hbm.at[sl], w1v)
            pltpu.sync_copy(w2_hbm.at[sl], w2v)
            c0 = pltpu.async_copy(x_hbm.at[i0v], r0v, sem)
            c1 = pltpu.async_copy(x_hbm.at[i1v], r1v, sem)
            c2 = pltpu.async_copy(x_hbm.at[i2v], r2v, sem)
            c0.wait()
            c1.wait()
            c2.wait()

            def row_body(r, _):
                w0s = w0v[r, :]
                w1s = w1v[r, :]
                w2s = w2v[r, :]
                for c in range(d // 16):
                    s = pl.ds(c * 16, 16)
                    outv[r, s] = (r0v[r, s] * w0s + r1v[r, s] * w1s
                                  + r2v[r, s] * w2s)
                return 0

            lax.fori_loop(0, batch_rows, row_body, 0)
            pltpu.sync_copy(outv, out_hbm.at[sl])
            return 0

        lax.fori_loop(0, nb, batch_body, 0)

    return interp(x, i0, i1, i2, w0, w1, w2)


def kernel(x, pos, pos_up, batch, batch_up):
    m = pos_up.shape[0]
    wn, idx, m_pad = _run_topk(pos, pos_up)
    i0 = idx[:, 0]
    i1 = idx[:, 1]
    i2 = idx[:, 2]
    m_pad = i0.shape[0]
    w0 = jnp.broadcast_to(wn[:, 0:1], (m_pad, 16))
    w1 = jnp.broadcast_to(wn[:, 1:2], (m_pad, 16))
    w2 = jnp.broadcast_to(wn[:, 2:3], (m_pad, 16))
    out = _interp_sc(x, i0, i1, i2, w0, w1, w2)
    return out[:m]
